# baseline (speedup 1.0000x reference)
import functools

import jax
import jax.numpy as jnp
from jax import lax
from jax.experimental import pallas as pl
from jax.experimental.pallas import tpu as pltpu
from jax.experimental.pallas import tpu_sc as plsc

NC = 2
NS = 16
NW = NC * NS
LN_EPS = 1e-5


def _sc_mesh(nc=NC):
    return plsc.VectorSubcoreMesh(
        core_axis_name="c", subcore_axis_name="s", num_cores=nc, num_subcores=NS
    )


def _make_gather(n_e, H, ch, chunk_base, nc=NC):
    per_w = n_e // (nc * NS)
    n_ch = per_w // ch
    grp = n_ch // 2
    tail = n_ch - 2 * grp

    @functools.partial(
        pl.kernel,
        mesh=_sc_mesh(nc),
        out_type=jax.ShapeDtypeStruct((n_e, H), jnp.float32),
        scratch_types=[
            pltpu.VMEM((2 * ch,), jnp.int32),
            pltpu.VMEM((2 * ch,), jnp.int32),
            pltpu.VMEM((2 * ch, H), jnp.float32),
            pltpu.VMEM((2 * ch, H), jnp.float32),
            pltpu.VMEM((ch, H), jnp.float32),
            pltpu.VMEM((ch, H), jnp.float32),
            pltpu.SemaphoreType.DMA,
            pltpu.SemaphoreType.DMA,
            pltpu.SemaphoreType.DMA,
            pltpu.SemaphoreType.DMA,
        ],
    )
    def gather(ab_hbm, idx_hbm, out_hbm, i0, i1, r0, r1, s0, s1, g0, g1, t0, t1):
        wid = lax.axis_index("s") * nc + lax.axis_index("c")
        cbase = chunk_base + wid * n_ch
        ibufs = (i0, i1)
        rbufs = (r0, r1)
        sbufs = (s0, s1)
        gsems = (g0, g1)
        ssems = (t0, t1)

        def issue(k, b):
            off = (cbase + k) * (2 * ch)
            pltpu.sync_copy(idx_hbm.at[pl.ds(off, 2 * ch)], ibufs[b])
            pltpu.async_copy(ab_hbm.at[ibufs[b]], rbufs[b], gsems[b])

        def gwait(b):
            pltpu.make_async_copy(ab_hbm.at[ibufs[b]], rbufs[b], gsems[b]).wait()

        def swait(b):
            pltpu.make_async_copy(
                sbufs[b], out_hbm.at[pl.ds(0, ch)], ssems[b]).wait()

        def add_store(k, b):
            def row(i, c2):
                for q in range(H // 16):
                    sl = pl.ds(q * 16, 16)
                    sbufs[b][i, sl] = rbufs[b][i, sl] + rbufs[b][ch + i, sl]
                return c2

            lax.fori_loop(0, ch, row, 0)
            pltpu.async_copy(
                sbufs[b], out_hbm.at[pl.ds(wid * per_w + k * ch, ch)], ssems[b])

        issue(0, 0)
        if n_ch > 1:
            issue(1, 1)

        def group(t, carry):
            for b in range(2):
                k = 2 * t + b
                gwait(b)

                @pl.when(t >= 1)
                def _():
                    swait(b)

                add_store(k, b)

                @pl.when(k + 2 < n_ch)
                def _():
                    issue(k + 2, b)

            return carry

        lax.fori_loop(0, grp, group, 0)
        if tail:
            gwait(0)
            if n_ch > 2:
                swait(0)
            add_store(n_ch - 1, 0)
        if n_ch > 1:
            swait(0)
            swait(1)
        else:
            swait(0)

    return gather


def _make_scatter(R, S, H, ch, nc=NC):
    per_w = R // (nc * NS)
    n_ch = per_w // ch
    z_full = S // ch
    z_tail = S % ch

    grp = n_ch // 2
    tail = n_ch - 2 * grp

    @functools.partial(
        pl.kernel,
        mesh=_sc_mesh(nc),
        out_type=jax.ShapeDtypeStruct((nc, S, H), jnp.float32),
        scratch_types=[
            pltpu.VMEM((ch,), jnp.int32),
            pltpu.VMEM((ch,), jnp.int32),
            pltpu.VMEM((ch, H), jnp.float32),
            pltpu.VMEM((ch, H), jnp.float32),
            pltpu.VMEM((ch, H), jnp.float32),
            pltpu.VMEM_SHARED((S, H), jnp.float32),
            pltpu.SemaphoreType.DMA,
            pltpu.SemaphoreType.DMA,
        ],
    )
    def scatter(src_hbm, idx_hbm, out_hbm, x0, x1, w0, w1, zbuf, shared, g0, g1):
        cid = lax.axis_index("c")
        sid = lax.axis_index("s")
        wid = sid * nc + cid
        base = wid * per_w
        ibufs = (x0, x1)
        rbufs = (w0, w1)
        gsems = (g0, g1)

        def zrow(i, c2):
            for q in range(H // 16):
                zbuf[i, pl.ds(q * 16, 16)] = jnp.zeros((16,), jnp.float32)
            return c2

        lax.fori_loop(0, ch, zrow, 0)

        def zchunk(z, c2):
            @pl.when(z % NS == sid)
            def _():
                pltpu.sync_copy(zbuf, shared.at[pl.ds(z * ch, ch)])

            return c2

        lax.fori_loop(0, z_full, zchunk, 0)
        if z_tail:
            @pl.when(z_full % NS == sid)
            def _():
                pltpu.sync_copy(zbuf.at[pl.ds(0, z_tail)],
                                shared.at[pl.ds(z_full * ch, z_tail)])

        plsc.subcore_barrier()

        def issue(k, b):
            off = base + k * ch
            pltpu.sync_copy(idx_hbm.at[pl.ds(off, ch)], ibufs[b])
            pltpu.async_copy(src_hbm.at[pl.ds(off, ch)], rbufs[b], gsems[b])

        def gwait(b):
            pltpu.make_async_copy(
                src_hbm.at[pl.ds(0, ch)], rbufs[b], gsems[b]).wait()

        issue(0, 0)
        if n_ch > 1:
            issue(1, 1)

        def sgroup(t, carry):
            for b in range(2):
                k = 2 * t + b
                gwait(b)
                pltpu.sync_copy(rbufs[b], shared.at[ibufs[b]], add=True)

                @pl.when(k + 2 < n_ch)
                def _():
                    issue(k + 2, b)

            return carry

        lax.fori_loop(0, grp, sgroup, 0)
        if tail:
            gwait(0)
            pltpu.sync_copy(rbufs[0], shared.at[ibufs[0]], add=True)
        plsc.subcore_barrier()

        def ochunk(z, c2):
            @pl.when(z % NS == sid)
            def _():
                pltpu.sync_copy(shared.at[pl.ds(z * ch, ch)],
                                out_hbm.at[cid, pl.ds(z * ch, ch)])

            return c2

        lax.fori_loop(0, z_full, ochunk, 0)
        if z_tail:
            @pl.when(z_full % NS == sid)
            def _():
                pltpu.sync_copy(shared.at[pl.ds(z_full * ch, z_tail)],
                                out_hbm.at[cid, pl.ds(z_full * ch, z_tail)])

    return scatter


def _dot(a, b):
    return jnp.dot(a.astype(jnp.bfloat16), b.astype(jnp.bfloat16),
                   preferred_element_type=jnp.float32)


def _ln(t, g, beta):
    mu = jnp.mean(t, axis=-1, keepdims=True)
    var = jnp.mean((t - mu) * (t - mu), axis=-1, keepdims=True)
    return (t - mu) * lax.rsqrt(var + LN_EPS) * g + beta


def _node_enc_call(x, ws, bs, g, beta, w1a, w1b):
    N, H = x.shape[0], ws[-1].shape[1]

    def body(x_ref, w0, w1, w2, w3, b0, b1, b2, b3, g_ref, bt_ref, wa, wb,
             h_ref, ab_ref):
        t = jnp.maximum(_dot(x_ref[...], w0[...]) + b0[...], 0.0)
        t = jnp.maximum(_dot(t, w1[...]) + b1[...], 0.0)
        t = jnp.maximum(_dot(t, w2[...]) + b2[...], 0.0)
        t = _dot(t, w3[...]) + b3[...]
        h = _ln(t, g_ref[...], bt_ref[...])
        h_ref[...] = h
        ab_ref[0:N, :] = _dot(h, wa[...])
        ab_ref[N:, :] = _dot(h, wb[...])

    out_shape = (jax.ShapeDtypeStruct((N, H), jnp.float32),
                 jax.ShapeDtypeStruct((2 * N, H), jnp.float32))
    return pl.pallas_call(body, out_shape=out_shape)(
        x, *ws, *bs, g, beta, w1a, w1b)


def _edge_enc_call(edge_attr, ws, bs, g, beta, te):
    E = edge_attr.shape[0]
    D = edge_attr.shape[1]
    H = ws[-1].shape[1]
    grid = (E // te,)

    def body(ea_ref, w0, w1, w2, w3, b0, b1, b2, b3, g_ref, bt_ref, out_ref):
        t = jnp.maximum(_dot(ea_ref[...], w0[...]) + b0[...], 0.0)
        t = jnp.maximum(_dot(t, w1[...]) + b1[...], 0.0)
        t = jnp.maximum(_dot(t, w2[...]) + b2[...], 0.0)
        t = _dot(t, w3[...]) + b3[...]
        out_ref[...] = _ln(t, g_ref[...], bt_ref[...])

    full = lambda a: pl.BlockSpec(a.shape, lambda i: (0,) * a.ndim)
    in_specs = [pl.BlockSpec((te, D), lambda i: (i, 0))]
    in_specs += [full(w) for w in ws] + [full(b) for b in bs] + [full(g), full(beta)]
    return pl.pallas_call(
        body,
        grid=grid,
        in_specs=in_specs,
        out_specs=pl.BlockSpec((te, H), lambda i: (i, 0)),
        out_shape=jax.ShapeDtypeStruct((E, H), jnp.float32),
    )(edge_attr, *ws, *bs, g, beta)


def _edge_mlp_call(G, e, w1c, ws, bs, g, beta, te):
    E, H = e.shape
    grid = (E // te,)

    def body(g_in, e_ref, wc, w1, w2, w3, b0, b1, b2, b3, gg, bt, out_ref):
        ev = e_ref[...]
        t = g_in[...] + _dot(ev, wc[...]) + b0[...]
        t = jnp.maximum(t, 0.0)
        t = jnp.maximum(_dot(t, w1[...]) + b1[...], 0.0)
        t = jnp.maximum(_dot(t, w2[...]) + b2[...], 0.0)
        t = _dot(t, w3[...]) + b3[...]
        out_ref[...] = _ln(t, gg[...], bt[...]) + ev

    full = lambda a: pl.BlockSpec(a.shape, lambda i: (0,) * a.ndim)
    row = pl.BlockSpec((te, H), lambda i: (i, 0))
    in_specs = [row, row, full(w1c)] + [full(w) for w in ws]
    in_specs += [full(b) for b in bs] + [full(g), full(beta)]
    return pl.pallas_call(
        body,
        grid=grid,
        in_specs=in_specs,
        out_specs=row,
        out_shape=jax.ShapeDtypeStruct((E, H), jnp.float32),
    )(G, e, w1c, *ws, *bs, g, beta)


def _node_block_call(h, p_a, p_b, v1a, v1b, ws, bs, g, beta, w1a, w1b):
    N, H = h.shape
    with_ab = w1a is not None

    def body(h_ref, pa_ref, pb_ref, va, vb, w1, w2, w3, b0, b1, b2, b3, gg, bt,
             *rest):
        hv = h_ref[...]
        agg = pa_ref[0] + pb_ref[0]
        for c in range(1, p_a.shape[0]):
            agg = agg + pa_ref[c] + pb_ref[c]
        t = _dot(hv, va[...]) + _dot(agg, vb[...]) + b0[...]
        t = jnp.maximum(t, 0.0)
        t = jnp.maximum(_dot(t, w1[...]) + b1[...], 0.0)
        t = jnp.maximum(_dot(t, w2[...]) + b2[...], 0.0)
        t = _dot(t, w3[...]) + b3[...]
        h_new = _ln(t, gg[...], bt[...]) + hv
        if with_ab:
            wa, wb, h_out, ab_out = rest
            h_out[...] = h_new
            ab_out[0:N, :] = _dot(h_new, wa[...])
            ab_out[N:, :] = _dot(h_new, wb[...])
        else:
            (h_out,) = rest
            h_out[...] = h_new

    out_shape = [jax.ShapeDtypeStruct((N, H), jnp.float32)]
    if with_ab:
        out_shape.append(jax.ShapeDtypeStruct((2 * N, H), jnp.float32))
    args = [h, p_a, p_b, v1a, v1b, *ws, *bs, g, beta]
    if with_ab:
        args += [w1a, w1b]
    res = pl.pallas_call(body, out_shape=tuple(out_shape))(*args)
    return res


def _pool_dec_call(h_pad, batch_row, ws, bs, w_last_row, b_last, n_graphs):
    NP = batch_row.shape[1]

    def body(h_ref, batch_ref, w0, w1, w2, b0, b1, b2, wl, bl, out_ref):
        ids = lax.broadcasted_iota(jnp.int32, (n_graphs, NP), 0)
        bm = jnp.broadcast_to(batch_ref[...], (n_graphs, NP))
        onehot = (bm == ids).astype(jnp.float32)
        counts = jnp.sum(onehot, axis=1, keepdims=True)
        sums = jnp.dot(onehot, h_ref[...],
                       preferred_element_type=jnp.float32)
        pooled = sums / jnp.maximum(counts, 1.0)
        t = jnp.maximum(_dot(pooled, w0[...]) + b0[...], 0.0)
        t = jnp.maximum(_dot(t, w1[...]) + b1[...], 0.0)
        t = jnp.maximum(_dot(t, w2[...]) + b2[...], 0.0)
        o = jnp.sum(t * wl[...], axis=-1, keepdims=True) + bl[...]
        out_ref[...] = jax.nn.sigmoid(o)

    return pl.pallas_call(
        body, out_shape=jax.ShapeDtypeStruct((n_graphs, 1), jnp.float32)
    )(h_pad, batch_row, *ws, *bs, w_last_row, b_last)


def _unpack_mlp(mlp):
    layers, ln = mlp
    ws = [W for W, _ in layers]
    bs = [b.reshape(1, -1) for _, b in layers]
    if ln is not None:
        g, beta = ln
        return ws, bs, g.reshape(1, -1), beta.reshape(1, -1)
    return ws, bs, None, None


def kernel(x, edge_index, edge_attr, batch, params):
    N, _ = x.shape
    E = edge_attr.shape[0]
    H = params["node_enc"][0][-1][0].shape[1]
    n_graphs = 16
    mp = len(params["blocks"])

    s = edge_index[0]
    r = edge_index[1]

    ne_ws, ne_bs, ne_g, ne_b = _unpack_mlp(params["node_enc"])
    ee_ws, ee_bs, ee_g, ee_b = _unpack_mlp(params["edge_enc"])
    blocks = []
    for blk in params["blocks"]:
        ews, ebs, eg, eb = _unpack_mlp(blk["edge"])
        w1 = ews[0]
        nws, nbs, ng, nb = _unpack_mlp(blk["node"])
        blocks.append(
            dict(
                w1a=w1[:H], w1b=w1[H:2 * H], w1c=w1[2 * H:],
                e_ws=ews[1:], e_bs=ebs, e_g=eg, e_b=eb,
                v1a=nws[0][:H], v1b=nws[0][H:],
                n_ws=nws[1:], n_bs=nbs, n_g=ng, n_b=nb,
            )
        )
    d_ws, d_bs, _, _ = _unpack_mlp(params["dec"])
    d_last_row = d_ws[-1].reshape(1, -1)
    d_blast = d_bs[-1].reshape(1, 1)

    ch = 80
    grain = NW * ch
    e1 = (E // (2 * grain)) * grain
    e2 = E - e1
    te1 = e1 // 32
    te2 = e2 // 32
    gather_a = _make_gather(e1, H, ch, 0, 1)
    gather_b = _make_gather(e2, H, ch, e1 // ch, 1)
    scat_a = _make_scatter(e1, N, H, ch, 1)
    scat_b = _make_scatter(e2, N, H, ch, 1)
    r_a, r_b = r[:e1], r[e1:]

    idx2 = jnp.stack([s.reshape(-1, ch), r.reshape(-1, ch) + N],
                     axis=1).reshape(-1)

    h, AB = _node_enc_call(x, ne_ws, ne_bs, ne_g, ne_b,
                           blocks[0]["w1a"], blocks[0]["w1b"])
    e_a = _edge_enc_call(edge_attr[:e1], ee_ws, ee_bs, ee_g, ee_b, te1)
    e_b = _edge_enc_call(edge_attr[e1:], ee_ws, ee_bs, ee_g, ee_b, te2)

    for k in range(mp):
        blk = blocks[k]
        G_a = gather_a(AB, idx2)
        G_b = gather_b(AB, idx2)
        e_a = _edge_mlp_call(G_a, e_a, blk["w1c"], blk["e_ws"], blk["e_bs"],
                             blk["e_g"], blk["e_b"], te1)
        p_a = scat_a(e_a, r_a)
        e_b = _edge_mlp_call(G_b, e_b, blk["w1c"], blk["e_ws"], blk["e_bs"],
                             blk["e_g"], blk["e_b"], te2)
        p_b = scat_b(e_b, r_b)
        last = k == mp - 1
        if last:
            (h,) = _node_block_call(h, p_a, p_b, blk["v1a"], blk["v1b"],
                                    blk["n_ws"], blk["n_bs"], blk["n_g"],
                                    blk["n_b"], None, None)
        else:
            nxt = blocks[k + 1]
            h, AB = _node_block_call(h, p_a, p_b, blk["v1a"], blk["v1b"],
                                     blk["n_ws"], blk["n_bs"], blk["n_g"],
                                     blk["n_b"], nxt["w1a"], nxt["w1b"])

    n_pad = ((N + 127) // 128) * 128
    h_pad = jnp.pad(h, ((0, n_pad - N), (0, 0)))
    batch_pad = jnp.pad(batch, (0, n_pad - N), constant_values=n_graphs)
    out = _pool_dec_call(h_pad, batch_pad.reshape(1, n_pad),
                         d_ws[:-1], d_bs[:-1], d_last_row, d_blast, n_graphs)
    return out


# speedup vs baseline: 1.3922x; 1.3922x over previous
import functools

import jax
import jax.numpy as jnp
from jax import lax
from jax.experimental import pallas as pl
from jax.experimental.pallas import tpu as pltpu
from jax.experimental.pallas import tpu_sc as plsc

NC = 2
NS = 16
NW = NC * NS
LN_EPS = 1e-5


def _sc_mesh(nc=NC):
    return plsc.VectorSubcoreMesh(
        core_axis_name="c", subcore_axis_name="s", num_cores=nc, num_subcores=NS
    )


def _make_gather(n_e, H, ch, chunk_base, nc=NC):
    per_w = n_e // (nc * NS)
    n_ch = per_w // ch
    grp = n_ch // 2
    tail = n_ch - 2 * grp

    @functools.partial(
        pl.kernel,
        mesh=_sc_mesh(nc),
        out_type=jax.ShapeDtypeStruct((n_e, H), jnp.float32),
        scratch_types=[
            pltpu.VMEM((2 * ch,), jnp.int32),
            pltpu.VMEM((2 * ch,), jnp.int32),
            pltpu.VMEM((2 * ch, H), jnp.float32),
            pltpu.VMEM((2 * ch, H), jnp.float32),
            pltpu.VMEM((ch, H), jnp.float32),
            pltpu.VMEM((ch, H), jnp.float32),
            pltpu.SemaphoreType.DMA,
            pltpu.SemaphoreType.DMA,
            pltpu.SemaphoreType.DMA,
            pltpu.SemaphoreType.DMA,
        ],
    )
    def gather(ab_hbm, idx_hbm, out_hbm, i0, i1, r0, r1, s0, s1, g0, g1, t0, t1):
        wid = lax.axis_index("s") * nc + lax.axis_index("c")
        cbase = chunk_base + wid * n_ch
        ibufs = (i0, i1)
        rbufs = (r0, r1)
        sbufs = (s0, s1)
        gsems = (g0, g1)
        ssems = (t0, t1)

        def issue(k, b):
            off = (cbase + k) * (2 * ch)
            pltpu.sync_copy(idx_hbm.at[pl.ds(off, 2 * ch)], ibufs[b])
            pltpu.async_copy(ab_hbm.at[ibufs[b]], rbufs[b], gsems[b])

        def gwait(b):
            pltpu.make_async_copy(ab_hbm.at[ibufs[b]], rbufs[b], gsems[b]).wait()

        def swait(b):
            pltpu.make_async_copy(
                sbufs[b], out_hbm.at[pl.ds(0, ch)], ssems[b]).wait()

        def add_store(k, b):
            def row(i, c2):
                for q in range(H // 16):
                    sl = pl.ds(q * 16, 16)
                    sbufs[b][i, sl] = rbufs[b][i, sl] + rbufs[b][ch + i, sl]
                return c2

            lax.fori_loop(0, ch, row, 0)
            pltpu.async_copy(
                sbufs[b], out_hbm.at[pl.ds(wid * per_w + k * ch, ch)], ssems[b])

        issue(0, 0)
        if n_ch > 1:
            issue(1, 1)

        def group(t, carry):
            for b in range(2):
                k = 2 * t + b
                gwait(b)

                @pl.when(t >= 1)
                def _():
                    swait(b)

                add_store(k, b)

                @pl.when(k + 2 < n_ch)
                def _():
                    issue(k + 2, b)

            return carry

        lax.fori_loop(0, grp, group, 0)
        if tail:
            gwait(0)
            if n_ch > 2:
                swait(0)
            add_store(n_ch - 1, 0)
        if n_ch > 1:
            swait(0)
            swait(1)
        else:
            swait(0)

    return gather


def _make_scatter(R, S, H, ch, nc=NC):
    per_w = R // (nc * NS)
    n_ch = per_w // ch
    z_full = S // ch
    z_tail = S % ch

    grp = n_ch // 2
    tail = n_ch - 2 * grp

    @functools.partial(
        pl.kernel,
        mesh=_sc_mesh(nc),
        out_type=jax.ShapeDtypeStruct((nc, S, H), jnp.float32),
        scratch_types=[
            pltpu.VMEM((ch,), jnp.int32),
            pltpu.VMEM((ch,), jnp.int32),
            pltpu.VMEM((ch, H), jnp.float32),
            pltpu.VMEM((ch, H), jnp.float32),
            pltpu.VMEM((ch, H), jnp.float32),
            pltpu.VMEM_SHARED((S, H), jnp.float32),
            pltpu.SemaphoreType.DMA,
            pltpu.SemaphoreType.DMA,
        ],
    )
    def scatter(src_hbm, idx_hbm, out_hbm, x0, x1, w0, w1, zbuf, shared, g0, g1):
        cid = lax.axis_index("c")
        sid = lax.axis_index("s")
        wid = sid * nc + cid
        base = wid * per_w
        ibufs = (x0, x1)
        rbufs = (w0, w1)
        gsems = (g0, g1)

        def zrow(i, c2):
            for q in range(H // 16):
                zbuf[i, pl.ds(q * 16, 16)] = jnp.zeros((16,), jnp.float32)
            return c2

        lax.fori_loop(0, ch, zrow, 0)

        def zchunk(z, c2):
            @pl.when(z % NS == sid)
            def _():
                pltpu.sync_copy(zbuf, shared.at[pl.ds(z * ch, ch)])

            return c2

        lax.fori_loop(0, z_full, zchunk, 0)
        if z_tail:
            @pl.when(z_full % NS == sid)
            def _():
                pltpu.sync_copy(zbuf.at[pl.ds(0, z_tail)],
                                shared.at[pl.ds(z_full * ch, z_tail)])

        plsc.subcore_barrier()

        def issue(k, b):
            off = base + k * ch
            pltpu.sync_copy(idx_hbm.at[pl.ds(off, ch)], ibufs[b])
            pltpu.async_copy(src_hbm.at[pl.ds(off, ch)], rbufs[b], gsems[b])

        def gwait(b):
            pltpu.make_async_copy(
                src_hbm.at[pl.ds(0, ch)], rbufs[b], gsems[b]).wait()

        issue(0, 0)
        if n_ch > 1:
            issue(1, 1)

        def sgroup(t, carry):
            for b in range(2):
                k = 2 * t + b
                gwait(b)
                pltpu.sync_copy(rbufs[b], shared.at[ibufs[b]], add=True)

                @pl.when(k + 2 < n_ch)
                def _():
                    issue(k + 2, b)

            return carry

        lax.fori_loop(0, grp, sgroup, 0)
        if tail:
            gwait(0)
            pltpu.sync_copy(rbufs[0], shared.at[ibufs[0]], add=True)
        plsc.subcore_barrier()

        def ochunk(z, c2):
            @pl.when(z % NS == sid)
            def _():
                pltpu.sync_copy(shared.at[pl.ds(z * ch, ch)],
                                out_hbm.at[cid, pl.ds(z * ch, ch)])

            return c2

        lax.fori_loop(0, z_full, ochunk, 0)
        if z_tail:
            @pl.when(z_full % NS == sid)
            def _():
                pltpu.sync_copy(shared.at[pl.ds(z_full * ch, z_tail)],
                                out_hbm.at[cid, pl.ds(z_full * ch, z_tail)])

    return scatter


def _dot(a, b):
    return jnp.dot(a.astype(jnp.bfloat16), b.astype(jnp.bfloat16),
                   preferred_element_type=jnp.float32)


def _ln(t, g, beta):
    mu = jnp.mean(t, axis=-1, keepdims=True)
    var = jnp.mean((t - mu) * (t - mu), axis=-1, keepdims=True)
    return (t - mu) * lax.rsqrt(var + LN_EPS) * g + beta


def _node_enc_call(x, ws, bs, g, beta, w1a, w1b):
    N, H = x.shape[0], ws[-1].shape[1]

    def body(x_ref, w0, w1, w2, w3, b0, b1, b2, b3, g_ref, bt_ref, wa, wb,
             h_ref, ab_ref):
        t = jnp.maximum(_dot(x_ref[...], w0[...]) + b0[...], 0.0)
        t = jnp.maximum(_dot(t, w1[...]) + b1[...], 0.0)
        t = jnp.maximum(_dot(t, w2[...]) + b2[...], 0.0)
        t = _dot(t, w3[...]) + b3[...]
        h = _ln(t, g_ref[...], bt_ref[...])
        h_ref[...] = h
        ab_ref[0:N, :] = _dot(h, wa[...])
        ab_ref[N:, :] = _dot(h, wb[...])

    out_shape = (jax.ShapeDtypeStruct((N, H), jnp.float32),
                 jax.ShapeDtypeStruct((2 * N, H), jnp.float32))
    return pl.pallas_call(body, out_shape=out_shape)(
        x, *ws, *bs, g, beta, w1a, w1b)


def _edge_enc_call(edge_attr, ws, bs, g, beta, te):
    E = edge_attr.shape[0]
    D = edge_attr.shape[1]
    H = ws[-1].shape[1]
    grid = (E // te,)

    def body(ea_ref, w0, w1, w2, w3, b0, b1, b2, b3, g_ref, bt_ref, out_ref):
        t = jnp.maximum(_dot(ea_ref[...], w0[...]) + b0[...], 0.0)
        t = jnp.maximum(_dot(t, w1[...]) + b1[...], 0.0)
        t = jnp.maximum(_dot(t, w2[...]) + b2[...], 0.0)
        t = _dot(t, w3[...]) + b3[...]
        out_ref[...] = _ln(t, g_ref[...], bt_ref[...])

    full = lambda a: pl.BlockSpec(a.shape, lambda i: (0,) * a.ndim)
    in_specs = [pl.BlockSpec((te, D), lambda i: (i, 0))]
    in_specs += [full(w) for w in ws] + [full(b) for b in bs] + [full(g), full(beta)]
    return pl.pallas_call(
        body,
        grid=grid,
        in_specs=in_specs,
        out_specs=pl.BlockSpec((te, H), lambda i: (i, 0)),
        out_shape=jax.ShapeDtypeStruct((E, H), jnp.float32),
    )(edge_attr, *ws, *bs, g, beta)


def _edge_mlp_call(G, e, w1c, ws, bs, g, beta, te):
    E, H = e.shape
    grid = (E // te,)

    def body(g_in, e_ref, wc, w1, w2, w3, b0, b1, b2, b3, gg, bt, out_ref):
        ev = e_ref[...]
        t = g_in[...] + _dot(ev, wc[...]) + b0[...]
        t = jnp.maximum(t, 0.0)
        t = jnp.maximum(_dot(t, w1[...]) + b1[...], 0.0)
        t = jnp.maximum(_dot(t, w2[...]) + b2[...], 0.0)
        t = _dot(t, w3[...]) + b3[...]
        out_ref[...] = _ln(t, gg[...], bt[...]) + ev

    full = lambda a: pl.BlockSpec(a.shape, lambda i: (0,) * a.ndim)
    row = pl.BlockSpec((te, H), lambda i: (i, 0))
    in_specs = [row, row, full(w1c)] + [full(w) for w in ws]
    in_specs += [full(b) for b in bs] + [full(g), full(beta)]
    return pl.pallas_call(
        body,
        grid=grid,
        in_specs=in_specs,
        out_specs=row,
        out_shape=jax.ShapeDtypeStruct((E, H), jnp.float32),
    )(G, e, w1c, *ws, *bs, g, beta)


def _node_block_call(h, p_a, p_b, v1a, v1b, ws, bs, g, beta, w1a, w1b):
    N, H = h.shape
    with_ab = w1a is not None

    def body(h_ref, pa_ref, pb_ref, va, vb, w1, w2, w3, b0, b1, b2, b3, gg, bt,
             *rest):
        hv = h_ref[...]
        agg = pa_ref[0] + pb_ref[0]
        for c in range(1, p_a.shape[0]):
            agg = agg + pa_ref[c] + pb_ref[c]
        t = _dot(hv, va[...]) + _dot(agg, vb[...]) + b0[...]
        t = jnp.maximum(t, 0.0)
        t = jnp.maximum(_dot(t, w1[...]) + b1[...], 0.0)
        t = jnp.maximum(_dot(t, w2[...]) + b2[...], 0.0)
        t = _dot(t, w3[...]) + b3[...]
        h_new = _ln(t, gg[...], bt[...]) + hv
        if with_ab:
            wa, wb, h_out, ab_out = rest
            h_out[...] = h_new
            ab_out[0:N, :] = _dot(h_new, wa[...])
            ab_out[N:, :] = _dot(h_new, wb[...])
        else:
            (h_out,) = rest
            h_out[...] = h_new

    out_shape = [jax.ShapeDtypeStruct((N, H), jnp.float32)]
    if with_ab:
        out_shape.append(jax.ShapeDtypeStruct((2 * N, H), jnp.float32))
    args = [h, p_a, p_b, v1a, v1b, *ws, *bs, g, beta]
    if with_ab:
        args += [w1a, w1b]
    res = pl.pallas_call(body, out_shape=tuple(out_shape))(*args)
    return res


def _pool_dec_call(h_pad, batch_row, ws, bs, w_last_row, b_last, n_graphs):
    NP = batch_row.shape[1]

    def body(h_ref, batch_ref, w0, w1, w2, b0, b1, b2, wl, bl, out_ref):
        ids = lax.broadcasted_iota(jnp.int32, (n_graphs, NP), 0)
        bm = jnp.broadcast_to(batch_ref[...], (n_graphs, NP))
        onehot = (bm == ids).astype(jnp.float32)
        counts = jnp.sum(onehot, axis=1, keepdims=True)
        sums = jnp.dot(onehot, h_ref[...],
                       preferred_element_type=jnp.float32)
        pooled = sums / jnp.maximum(counts, 1.0)
        t = jnp.maximum(_dot(pooled, w0[...]) + b0[...], 0.0)
        t = jnp.maximum(_dot(t, w1[...]) + b1[...], 0.0)
        t = jnp.maximum(_dot(t, w2[...]) + b2[...], 0.0)
        o = jnp.sum(t * wl[...], axis=-1, keepdims=True) + bl[...]
        out_ref[...] = jax.nn.sigmoid(o)

    return pl.pallas_call(
        body, out_shape=jax.ShapeDtypeStruct((n_graphs, 1), jnp.float32)
    )(h_pad, batch_row, *ws, *bs, w_last_row, b_last)


def _unpack_mlp(mlp):
    layers, ln = mlp
    ws = [W for W, _ in layers]
    bs = [b.reshape(1, -1) for _, b in layers]
    if ln is not None:
        g, beta = ln
        return ws, bs, g.reshape(1, -1), beta.reshape(1, -1)
    return ws, bs, None, None


def kernel(x, edge_index, edge_attr, batch, params):
    N, _ = x.shape
    E = edge_attr.shape[0]
    H = params["node_enc"][0][-1][0].shape[1]
    n_graphs = 16
    mp = len(params["blocks"])

    s = edge_index[0]
    r = edge_index[1]

    ne_ws, ne_bs, ne_g, ne_b = _unpack_mlp(params["node_enc"])
    ee_ws, ee_bs, ee_g, ee_b = _unpack_mlp(params["edge_enc"])
    blocks = []
    for blk in params["blocks"]:
        ews, ebs, eg, eb = _unpack_mlp(blk["edge"])
        w1 = ews[0]
        nws, nbs, ng, nb = _unpack_mlp(blk["node"])
        blocks.append(
            dict(
                w1a=w1[:H], w1b=w1[H:2 * H], w1c=w1[2 * H:],
                e_ws=ews[1:], e_bs=ebs, e_g=eg, e_b=eb,
                v1a=nws[0][:H], v1b=nws[0][H:],
                n_ws=nws[1:], n_bs=nbs, n_g=ng, n_b=nb,
            )
        )
    d_ws, d_bs, _, _ = _unpack_mlp(params["dec"])
    d_last_row = d_ws[-1].reshape(1, -1)
    d_blast = d_bs[-1].reshape(1, 1)

    ch = 80
    grain = NW * ch
    e1 = (E // (2 * grain)) * grain
    e2 = E - e1
    te1 = e1 // 32
    te2 = e2 // 32
    gather_a = _make_gather(e1, H, ch, 0)
    gather_b = _make_gather(e2, H, ch, e1 // ch)
    scat_a = _make_scatter(e1, N, H, ch)
    scat_b = _make_scatter(e2, N, H, ch)
    r_a, r_b = r[:e1], r[e1:]

    idx2 = jnp.stack([s.reshape(-1, ch), r.reshape(-1, ch) + N],
                     axis=1).reshape(-1)

    h, AB = _node_enc_call(x, ne_ws, ne_bs, ne_g, ne_b,
                           blocks[0]["w1a"], blocks[0]["w1b"])
    e_a = _edge_enc_call(edge_attr[:e1], ee_ws, ee_bs, ee_g, ee_b, te1)
    e_b = _edge_enc_call(edge_attr[e1:], ee_ws, ee_bs, ee_g, ee_b, te2)

    for k in range(mp):
        blk = blocks[k]
        G_a = gather_a(AB, idx2)
        G_b = gather_b(AB, idx2)
        e_a = _edge_mlp_call(G_a, e_a, blk["w1c"], blk["e_ws"], blk["e_bs"],
                             blk["e_g"], blk["e_b"], te1)
        p_a = scat_a(e_a, r_a)
        e_b = _edge_mlp_call(G_b, e_b, blk["w1c"], blk["e_ws"], blk["e_bs"],
                             blk["e_g"], blk["e_b"], te2)
        p_b = scat_b(e_b, r_b)
        last = k == mp - 1
        if last:
            (h,) = _node_block_call(h, p_a, p_b, blk["v1a"], blk["v1b"],
                                    blk["n_ws"], blk["n_bs"], blk["n_g"],
                                    blk["n_b"], None, None)
        else:
            nxt = blocks[k + 1]
            h, AB = _node_block_call(h, p_a, p_b, blk["v1a"], blk["v1b"],
                                     blk["n_ws"], blk["n_bs"], blk["n_g"],
                                     blk["n_b"], nxt["w1a"], nxt["w1b"])

    n_pad = ((N + 127) // 128) * 128
    h_pad = jnp.pad(h, ((0, n_pad - N), (0, 0)))
    batch_pad = jnp.pad(batch, (0, n_pad - N), constant_values=n_graphs)
    out = _pool_dec_call(h_pad, batch_pad.reshape(1, n_pad),
                         d_ws[:-1], d_bs[:-1], d_last_row, d_blast, n_graphs)
    return out


# speedup vs baseline: 1.3958x; 1.0026x over previous
import functools

import jax
import jax.numpy as jnp
from jax import lax
from jax.experimental import pallas as pl
from jax.experimental.pallas import tpu as pltpu
from jax.experimental.pallas import tpu_sc as plsc

NC = 2
NS = 16
NW = NC * NS
LN_EPS = 1e-5


def _sc_mesh(nc=NC):
    return plsc.VectorSubcoreMesh(
        core_axis_name="c", subcore_axis_name="s", num_cores=nc, num_subcores=NS
    )


def _make_gather(n_e, H, ch, chunk_base, nc=NC):
    per_w = n_e // (nc * NS)
    n_ch = per_w // ch
    grp = n_ch // 2
    tail = n_ch - 2 * grp

    @functools.partial(
        pl.kernel,
        mesh=_sc_mesh(nc),
        out_type=jax.ShapeDtypeStruct((n_e, H), jnp.float32),
        scratch_types=[
            pltpu.VMEM((2 * ch,), jnp.int32),
            pltpu.VMEM((2 * ch,), jnp.int32),
            pltpu.VMEM((2 * ch, H), jnp.float32),
            pltpu.VMEM((2 * ch, H), jnp.float32),
            pltpu.VMEM((ch, H), jnp.float32),
            pltpu.VMEM((ch, H), jnp.float32),
            pltpu.SemaphoreType.DMA,
            pltpu.SemaphoreType.DMA,
            pltpu.SemaphoreType.DMA,
            pltpu.SemaphoreType.DMA,
        ],
    )
    def gather(ab_hbm, idx_hbm, out_hbm, i0, i1, r0, r1, s0, s1, g0, g1, t0, t1):
        wid = lax.axis_index("s") * nc + lax.axis_index("c")
        cbase = chunk_base + wid * n_ch
        ibufs = (i0, i1)
        rbufs = (r0, r1)
        sbufs = (s0, s1)
        gsems = (g0, g1)
        ssems = (t0, t1)

        def issue(k, b):
            off = (cbase + k) * (2 * ch)
            pltpu.sync_copy(idx_hbm.at[pl.ds(off, 2 * ch)], ibufs[b])
            pltpu.async_copy(ab_hbm.at[ibufs[b]], rbufs[b], gsems[b])

        def gwait(b):
            pltpu.make_async_copy(ab_hbm.at[ibufs[b]], rbufs[b], gsems[b]).wait()

        def swait(b):
            pltpu.make_async_copy(
                sbufs[b], out_hbm.at[pl.ds(0, ch)], ssems[b]).wait()

        def add_store(k, b):
            def row(i, c2):
                for q in range(H // 16):
                    sl = pl.ds(q * 16, 16)
                    sbufs[b][i, sl] = rbufs[b][i, sl] + rbufs[b][ch + i, sl]
                return c2

            lax.fori_loop(0, ch, row, 0)
            pltpu.async_copy(
                sbufs[b], out_hbm.at[pl.ds(wid * per_w + k * ch, ch)], ssems[b])

        issue(0, 0)
        if n_ch > 1:
            issue(1, 1)

        def group(t, carry):
            for b in range(2):
                k = 2 * t + b
                gwait(b)

                @pl.when(t >= 1)
                def _():
                    swait(b)

                add_store(k, b)

                @pl.when(k + 2 < n_ch)
                def _():
                    issue(k + 2, b)

            return carry

        lax.fori_loop(0, grp, group, 0)
        if tail:
            gwait(0)
            if n_ch > 2:
                swait(0)
            add_store(n_ch - 1, 0)
        if n_ch > 1:
            swait(0)
            swait(1)
        else:
            swait(0)

    return gather


def _make_scatter(R, S, H, ch, nc=NC):
    per_w = R // (nc * NS)
    n_ch = per_w // ch
    z_full = S // ch
    z_tail = S % ch

    grp = n_ch // 2
    tail = n_ch - 2 * grp

    @functools.partial(
        pl.kernel,
        mesh=_sc_mesh(nc),
        out_type=jax.ShapeDtypeStruct((nc, S, H), jnp.float32),
        scratch_types=[
            pltpu.VMEM((ch,), jnp.int32),
            pltpu.VMEM((ch,), jnp.int32),
            pltpu.VMEM((ch, H), jnp.float32),
            pltpu.VMEM((ch, H), jnp.float32),
            pltpu.VMEM((ch, H), jnp.float32),
            pltpu.VMEM_SHARED((S, H), jnp.float32),
            pltpu.SemaphoreType.DMA,
            pltpu.SemaphoreType.DMA,
        ],
    )
    def scatter(src_hbm, idx_hbm, out_hbm, x0, x1, w0, w1, zbuf, shared, g0, g1):
        cid = lax.axis_index("c")
        sid = lax.axis_index("s")
        wid = sid * nc + cid
        base = wid * per_w
        ibufs = (x0, x1)
        rbufs = (w0, w1)
        gsems = (g0, g1)

        def issue(k, b):
            off = base + k * ch
            pltpu.sync_copy(idx_hbm.at[pl.ds(off, ch)], ibufs[b])
            pltpu.async_copy(src_hbm.at[pl.ds(off, ch)], rbufs[b], gsems[b])

        def gwait(b):
            pltpu.make_async_copy(
                src_hbm.at[pl.ds(0, ch)], rbufs[b], gsems[b]).wait()

        issue(0, 0)
        if n_ch > 1:
            issue(1, 1)

        def zrow(i, c2):
            for q in range(H // 16):
                zbuf[i, pl.ds(q * 16, 16)] = jnp.zeros((16,), jnp.float32)
            return c2

        lax.fori_loop(0, ch, zrow, 0)

        def zchunk(z, c2):
            @pl.when(z % NS == sid)
            def _():
                pltpu.sync_copy(zbuf, shared.at[pl.ds(z * ch, ch)])

            return c2

        lax.fori_loop(0, z_full, zchunk, 0)
        if z_tail:
            @pl.when(z_full % NS == sid)
            def _():
                pltpu.sync_copy(zbuf.at[pl.ds(0, z_tail)],
                                shared.at[pl.ds(z_full * ch, z_tail)])

        plsc.subcore_barrier()

        def sgroup(t, carry):
            for b in range(2):
                k = 2 * t + b
                gwait(b)
                pltpu.sync_copy(rbufs[b], shared.at[ibufs[b]], add=True)

                @pl.when(k + 2 < n_ch)
                def _():
                    issue(k + 2, b)

            return carry

        lax.fori_loop(0, grp, sgroup, 0)
        if tail:
            gwait(0)
            pltpu.sync_copy(rbufs[0], shared.at[ibufs[0]], add=True)
        plsc.subcore_barrier()

        def ochunk(z, c2):
            @pl.when(z % NS == sid)
            def _():
                pltpu.sync_copy(shared.at[pl.ds(z * ch, ch)],
                                out_hbm.at[cid, pl.ds(z * ch, ch)])

            return c2

        lax.fori_loop(0, z_full, ochunk, 0)
        if z_tail:
            @pl.when(z_full % NS == sid)
            def _():
                pltpu.sync_copy(shared.at[pl.ds(z_full * ch, z_tail)],
                                out_hbm.at[cid, pl.ds(z_full * ch, z_tail)])

    return scatter


def _dot(a, b):
    return jnp.dot(a.astype(jnp.bfloat16), b.astype(jnp.bfloat16),
                   preferred_element_type=jnp.float32)


def _ln(t, g, beta):
    mu = jnp.mean(t, axis=-1, keepdims=True)
    var = jnp.mean((t - mu) * (t - mu), axis=-1, keepdims=True)
    return (t - mu) * lax.rsqrt(var + LN_EPS) * g + beta


def _node_enc_call(x, ws, bs, g, beta, w1a, w1b):
    N, H = x.shape[0], ws[-1].shape[1]

    def body(x_ref, w0, w1, w2, w3, b0, b1, b2, b3, g_ref, bt_ref, wa, wb,
             h_ref, ab_ref):
        t = jnp.maximum(_dot(x_ref[...], w0[...]) + b0[...], 0.0)
        t = jnp.maximum(_dot(t, w1[...]) + b1[...], 0.0)
        t = jnp.maximum(_dot(t, w2[...]) + b2[...], 0.0)
        t = _dot(t, w3[...]) + b3[...]
        h = _ln(t, g_ref[...], bt_ref[...])
        h_ref[...] = h
        ab_ref[0:N, :] = _dot(h, wa[...])
        ab_ref[N:, :] = _dot(h, wb[...])

    out_shape = (jax.ShapeDtypeStruct((N, H), jnp.float32),
                 jax.ShapeDtypeStruct((2 * N, H), jnp.float32))
    return pl.pallas_call(body, out_shape=out_shape)(
        x, *ws, *bs, g, beta, w1a, w1b)


def _edge_enc_call(edge_attr, ws, bs, g, beta, te):
    E = edge_attr.shape[0]
    D = edge_attr.shape[1]
    H = ws[-1].shape[1]
    grid = (E // te,)

    def body(ea_ref, w0, w1, w2, w3, b0, b1, b2, b3, g_ref, bt_ref, out_ref):
        t = jnp.maximum(_dot(ea_ref[...], w0[...]) + b0[...], 0.0)
        t = jnp.maximum(_dot(t, w1[...]) + b1[...], 0.0)
        t = jnp.maximum(_dot(t, w2[...]) + b2[...], 0.0)
        t = _dot(t, w3[...]) + b3[...]
        out_ref[...] = _ln(t, g_ref[...], bt_ref[...])

    full = lambda a: pl.BlockSpec(a.shape, lambda i: (0,) * a.ndim)
    in_specs = [pl.BlockSpec((te, D), lambda i: (i, 0))]
    in_specs += [full(w) for w in ws] + [full(b) for b in bs] + [full(g), full(beta)]
    return pl.pallas_call(
        body,
        grid=grid,
        in_specs=in_specs,
        out_specs=pl.BlockSpec((te, H), lambda i: (i, 0)),
        out_shape=jax.ShapeDtypeStruct((E, H), jnp.float32),
    )(edge_attr, *ws, *bs, g, beta)


def _edge_mlp_call(G, e, w1c, ws, bs, g, beta, te):
    E, H = e.shape
    grid = (E // te,)

    def body(g_in, e_ref, wc, w1, w2, w3, b0, b1, b2, b3, gg, bt, out_ref):
        ev = e_ref[...]
        t = g_in[...] + _dot(ev, wc[...]) + b0[...]
        t = jnp.maximum(t, 0.0)
        t = jnp.maximum(_dot(t, w1[...]) + b1[...], 0.0)
        t = jnp.maximum(_dot(t, w2[...]) + b2[...], 0.0)
        t = _dot(t, w3[...]) + b3[...]
        out_ref[...] = _ln(t, gg[...], bt[...]) + ev

    full = lambda a: pl.BlockSpec(a.shape, lambda i: (0,) * a.ndim)
    row = pl.BlockSpec((te, H), lambda i: (i, 0))
    in_specs = [row, row, full(w1c)] + [full(w) for w in ws]
    in_specs += [full(b) for b in bs] + [full(g), full(beta)]
    return pl.pallas_call(
        body,
        grid=grid,
        in_specs=in_specs,
        out_specs=row,
        out_shape=jax.ShapeDtypeStruct((E, H), jnp.float32),
    )(G, e, w1c, *ws, *bs, g, beta)


def _node_block_call(h, p_a, p_b, v1a, v1b, ws, bs, g, beta, w1a, w1b):
    N, H = h.shape
    with_ab = w1a is not None

    def body(h_ref, pa_ref, pb_ref, va, vb, w1, w2, w3, b0, b1, b2, b3, gg, bt,
             *rest):
        hv = h_ref[...]
        agg = pa_ref[0] + pb_ref[0]
        for c in range(1, p_a.shape[0]):
            agg = agg + pa_ref[c] + pb_ref[c]
        t = _dot(hv, va[...]) + _dot(agg, vb[...]) + b0[...]
        t = jnp.maximum(t, 0.0)
        t = jnp.maximum(_dot(t, w1[...]) + b1[...], 0.0)
        t = jnp.maximum(_dot(t, w2[...]) + b2[...], 0.0)
        t = _dot(t, w3[...]) + b3[...]
        h_new = _ln(t, gg[...], bt[...]) + hv
        if with_ab:
            wa, wb, h_out, ab_out = rest
            h_out[...] = h_new
            ab_out[0:N, :] = _dot(h_new, wa[...])
            ab_out[N:, :] = _dot(h_new, wb[...])
        else:
            (h_out,) = rest
            h_out[...] = h_new

    out_shape = [jax.ShapeDtypeStruct((N, H), jnp.float32)]
    if with_ab:
        out_shape.append(jax.ShapeDtypeStruct((2 * N, H), jnp.float32))
    args = [h, p_a, p_b, v1a, v1b, *ws, *bs, g, beta]
    if with_ab:
        args += [w1a, w1b]
    res = pl.pallas_call(body, out_shape=tuple(out_shape))(*args)
    return res


def _pool_dec_call(h_pad, batch_row, ws, bs, w_last_row, b_last, n_graphs):
    NP = batch_row.shape[1]

    def body(h_ref, batch_ref, w0, w1, w2, b0, b1, b2, wl, bl, out_ref):
        ids = lax.broadcasted_iota(jnp.int32, (n_graphs, NP), 0)
        bm = jnp.broadcast_to(batch_ref[...], (n_graphs, NP))
        onehot = (bm == ids).astype(jnp.float32)
        counts = jnp.sum(onehot, axis=1, keepdims=True)
        sums = jnp.dot(onehot, h_ref[...],
                       preferred_element_type=jnp.float32)
        pooled = sums / jnp.maximum(counts, 1.0)
        t = jnp.maximum(_dot(pooled, w0[...]) + b0[...], 0.0)
        t = jnp.maximum(_dot(t, w1[...]) + b1[...], 0.0)
        t = jnp.maximum(_dot(t, w2[...]) + b2[...], 0.0)
        o = jnp.sum(t * wl[...], axis=-1, keepdims=True) + bl[...]
        out_ref[...] = jax.nn.sigmoid(o)

    return pl.pallas_call(
        body, out_shape=jax.ShapeDtypeStruct((n_graphs, 1), jnp.float32)
    )(h_pad, batch_row, *ws, *bs, w_last_row, b_last)


def _unpack_mlp(mlp):
    layers, ln = mlp
    ws = [W for W, _ in layers]
    bs = [b.reshape(1, -1) for _, b in layers]
    if ln is not None:
        g, beta = ln
        return ws, bs, g.reshape(1, -1), beta.reshape(1, -1)
    return ws, bs, None, None


def kernel(x, edge_index, edge_attr, batch, params):
    N, _ = x.shape
    E = edge_attr.shape[0]
    H = params["node_enc"][0][-1][0].shape[1]
    n_graphs = 16
    mp = len(params["blocks"])

    s = edge_index[0]
    r = edge_index[1]

    ne_ws, ne_bs, ne_g, ne_b = _unpack_mlp(params["node_enc"])
    ee_ws, ee_bs, ee_g, ee_b = _unpack_mlp(params["edge_enc"])
    blocks = []
    for blk in params["blocks"]:
        ews, ebs, eg, eb = _unpack_mlp(blk["edge"])
        w1 = ews[0]
        nws, nbs, ng, nb = _unpack_mlp(blk["node"])
        blocks.append(
            dict(
                w1a=w1[:H], w1b=w1[H:2 * H], w1c=w1[2 * H:],
                e_ws=ews[1:], e_bs=ebs, e_g=eg, e_b=eb,
                v1a=nws[0][:H], v1b=nws[0][H:],
                n_ws=nws[1:], n_bs=nbs, n_g=ng, n_b=nb,
            )
        )
    d_ws, d_bs, _, _ = _unpack_mlp(params["dec"])
    d_last_row = d_ws[-1].reshape(1, -1)
    d_blast = d_bs[-1].reshape(1, 1)

    ch = 80
    grain = NW * ch
    e1 = (E // (2 * grain)) * grain
    e2 = E - e1
    te1 = e1 // 32
    te2 = e2 // 32
    gather_a = _make_gather(e1, H, ch, 0)
    gather_b = _make_gather(e2, H, ch, e1 // ch)
    scat_a = _make_scatter(e1, N, H, ch)
    scat_b = _make_scatter(e2, N, H, ch)
    r_a, r_b = r[:e1], r[e1:]

    idx2 = jnp.stack([s.reshape(-1, ch), r.reshape(-1, ch) + N],
                     axis=1).reshape(-1)

    h, AB = _node_enc_call(x, ne_ws, ne_bs, ne_g, ne_b,
                           blocks[0]["w1a"], blocks[0]["w1b"])
    e_a = _edge_enc_call(edge_attr[:e1], ee_ws, ee_bs, ee_g, ee_b, te1)
    e_b = _edge_enc_call(edge_attr[e1:], ee_ws, ee_bs, ee_g, ee_b, te2)

    for k in range(mp):
        blk = blocks[k]
        G_a = gather_a(AB, idx2)
        G_b = gather_b(AB, idx2)
        e_a = _edge_mlp_call(G_a, e_a, blk["w1c"], blk["e_ws"], blk["e_bs"],
                             blk["e_g"], blk["e_b"], te1)
        p_a = scat_a(e_a, r_a)
        e_b = _edge_mlp_call(G_b, e_b, blk["w1c"], blk["e_ws"], blk["e_bs"],
                             blk["e_g"], blk["e_b"], te2)
        p_b = scat_b(e_b, r_b)
        last = k == mp - 1
        if last:
            (h,) = _node_block_call(h, p_a, p_b, blk["v1a"], blk["v1b"],
                                    blk["n_ws"], blk["n_bs"], blk["n_g"],
                                    blk["n_b"], None, None)
        else:
            nxt = blocks[k + 1]
            h, AB = _node_block_call(h, p_a, p_b, blk["v1a"], blk["v1b"],
                                     blk["n_ws"], blk["n_bs"], blk["n_g"],
                                     blk["n_b"], nxt["w1a"], nxt["w1b"])

    n_pad = ((N + 127) // 128) * 128
    h_pad = jnp.pad(h, ((0, n_pad - N), (0, 0)))
    batch_pad = jnp.pad(batch, (0, n_pad - N), constant_values=n_graphs)
    out = _pool_dec_call(h_pad, batch_pad.reshape(1, n_pad),
                         d_ws[:-1], d_bs[:-1], d_last_row, d_blast, n_graphs)
    return out


# speedup vs baseline: 1.4082x; 1.0088x over previous
import functools

import jax
import jax.numpy as jnp
from jax import lax
from jax.experimental import pallas as pl
from jax.experimental.pallas import tpu as pltpu
from jax.experimental.pallas import tpu_sc as plsc

NC = 2
NS = 16
NW = NC * NS
LN_EPS = 1e-5


def _sc_mesh(nc=NC):
    return plsc.VectorSubcoreMesh(
        core_axis_name="c", subcore_axis_name="s", num_cores=nc, num_subcores=NS
    )


def _make_gather(n_e, H, ch, chunk_base, nc=NC):
    per_w = n_e // (nc * NS)
    n_ch = per_w // ch
    grp = n_ch // 2
    tail = n_ch - 2 * grp

    @functools.partial(
        pl.kernel,
        mesh=_sc_mesh(nc),
        out_type=jax.ShapeDtypeStruct((n_e, H), jnp.float32),
        scratch_types=[
            pltpu.VMEM((2 * ch,), jnp.int32),
            pltpu.VMEM((2 * ch,), jnp.int32),
            pltpu.VMEM((2 * ch, H), jnp.float32),
            pltpu.VMEM((2 * ch, H), jnp.float32),
            pltpu.VMEM((ch, H), jnp.float32),
            pltpu.VMEM((ch, H), jnp.float32),
            pltpu.SemaphoreType.DMA,
            pltpu.SemaphoreType.DMA,
            pltpu.SemaphoreType.DMA,
            pltpu.SemaphoreType.DMA,
        ],
    )
    def gather(ab_hbm, idx_hbm, out_hbm, i0, i1, r0, r1, s0, s1, g0, g1, t0, t1):
        wid = lax.axis_index("s") * nc + lax.axis_index("c")
        cbase = chunk_base + wid * n_ch
        ibufs = (i0, i1)
        rbufs = (r0, r1)
        sbufs = (s0, s1)
        gsems = (g0, g1)
        ssems = (t0, t1)

        def issue(k, b):
            off = (cbase + k) * (2 * ch)
            pltpu.sync_copy(idx_hbm.at[pl.ds(off, 2 * ch)], ibufs[b])
            pltpu.async_copy(ab_hbm.at[ibufs[b]], rbufs[b], gsems[b])

        def gwait(b):
            pltpu.make_async_copy(ab_hbm.at[ibufs[b]], rbufs[b], gsems[b]).wait()

        def swait(b):
            pltpu.make_async_copy(
                sbufs[b], out_hbm.at[pl.ds(0, ch)], ssems[b]).wait()

        def add_store(k, b):
            def row(i, c2):
                for q in range(H // 16):
                    sl = pl.ds(q * 16, 16)
                    sbufs[b][i, sl] = rbufs[b][i, sl] + rbufs[b][ch + i, sl]
                return c2

            lax.fori_loop(0, ch, row, 0)
            pltpu.async_copy(
                sbufs[b], out_hbm.at[pl.ds(wid * per_w + k * ch, ch)], ssems[b])

        issue(0, 0)
        if n_ch > 1:
            issue(1, 1)

        def group(t, carry):
            for b in range(2):
                k = 2 * t + b
                gwait(b)

                @pl.when(t >= 1)
                def _():
                    swait(b)

                add_store(k, b)

                @pl.when(k + 2 < n_ch)
                def _():
                    issue(k + 2, b)

            return carry

        lax.fori_loop(0, grp, group, 0)
        if tail:
            gwait(0)
            if n_ch > 2:
                swait(0)
            add_store(n_ch - 1, 0)
        if n_ch > 1:
            swait(0)
            swait(1)
        else:
            swait(0)

    return gather


def _make_scatter(R, S, H, ch, nc=NC):
    per_w = R // (nc * NS)
    n_ch = per_w // ch
    z_full = S // ch
    z_tail = S % ch

    grp = n_ch // 2
    tail = n_ch - 2 * grp

    @functools.partial(
        pl.kernel,
        mesh=_sc_mesh(nc),
        out_type=jax.ShapeDtypeStruct((nc, S, H), jnp.float32),
        scratch_types=[
            pltpu.VMEM((ch,), jnp.int32),
            pltpu.VMEM((ch,), jnp.int32),
            pltpu.VMEM((ch, H), jnp.float32),
            pltpu.VMEM((ch, H), jnp.float32),
            pltpu.VMEM((ch, H), jnp.float32),
            pltpu.VMEM_SHARED((S, H), jnp.float32),
            pltpu.SemaphoreType.DMA,
            pltpu.SemaphoreType.DMA,
        ],
    )
    def scatter(src_hbm, idx_hbm, out_hbm, x0, x1, w0, w1, zbuf, shared, g0, g1):
        cid = lax.axis_index("c")
        sid = lax.axis_index("s")
        wid = sid * nc + cid
        base = wid * per_w
        ibufs = (x0, x1)
        rbufs = (w0, w1)
        gsems = (g0, g1)

        def issue(k, b):
            off = base + k * ch
            pltpu.sync_copy(idx_hbm.at[pl.ds(off, ch)], ibufs[b])
            pltpu.async_copy(src_hbm.at[pl.ds(off, ch)], rbufs[b], gsems[b])

        def gwait(b):
            pltpu.make_async_copy(
                src_hbm.at[pl.ds(0, ch)], rbufs[b], gsems[b]).wait()

        issue(0, 0)
        if n_ch > 1:
            issue(1, 1)

        def zrow(i, c2):
            for q in range(H // 16):
                zbuf[i, pl.ds(q * 16, 16)] = jnp.zeros((16,), jnp.float32)
            return c2

        lax.fori_loop(0, ch, zrow, 0)

        def zchunk(z, c2):
            @pl.when(z % NS == sid)
            def _():
                pltpu.sync_copy(zbuf, shared.at[pl.ds(z * ch, ch)])

            return c2

        lax.fori_loop(0, z_full, zchunk, 0)
        if z_tail:
            @pl.when(z_full % NS == sid)
            def _():
                pltpu.sync_copy(zbuf.at[pl.ds(0, z_tail)],
                                shared.at[pl.ds(z_full * ch, z_tail)])

        plsc.subcore_barrier()

        def sgroup(t, carry):
            for b in range(2):
                k = 2 * t + b
                gwait(b)
                pltpu.sync_copy(rbufs[b], shared.at[ibufs[b]], add=True)

                @pl.when(k + 2 < n_ch)
                def _():
                    issue(k + 2, b)

            return carry

        lax.fori_loop(0, grp, sgroup, 0)
        if tail:
            gwait(0)
            pltpu.sync_copy(rbufs[0], shared.at[ibufs[0]], add=True)
        plsc.subcore_barrier()

        def ochunk(z, c2):
            @pl.when(z % NS == sid)
            def _():
                pltpu.sync_copy(shared.at[pl.ds(z * ch, ch)],
                                out_hbm.at[cid, pl.ds(z * ch, ch)])

            return c2

        lax.fori_loop(0, z_full, ochunk, 0)
        if z_tail:
            @pl.when(z_full % NS == sid)
            def _():
                pltpu.sync_copy(shared.at[pl.ds(z_full * ch, z_tail)],
                                out_hbm.at[cid, pl.ds(z_full * ch, z_tail)])

    return scatter


def _dot(a, b):
    return jnp.dot(a.astype(jnp.bfloat16), b.astype(jnp.bfloat16),
                   preferred_element_type=jnp.float32)


def _ln(t, g, beta):
    mu = jnp.mean(t, axis=-1, keepdims=True)
    var = jnp.mean((t - mu) * (t - mu), axis=-1, keepdims=True)
    return (t - mu) * lax.rsqrt(var + LN_EPS) * g + beta


def _node_enc_call(x, ws, bs, g, beta, w1a, w1b):
    N, H = x.shape[0], ws[-1].shape[1]

    def body(x_ref, w0, w1, w2, w3, b0, b1, b2, b3, g_ref, bt_ref, wa, wb,
             h_ref, ab_ref):
        t = jnp.maximum(_dot(x_ref[...], w0[...]) + b0[...], 0.0)
        t = jnp.maximum(_dot(t, w1[...]) + b1[...], 0.0)
        t = jnp.maximum(_dot(t, w2[...]) + b2[...], 0.0)
        t = _dot(t, w3[...]) + b3[...]
        h = _ln(t, g_ref[...], bt_ref[...])
        h_ref[...] = h
        ab_ref[0:N, :] = _dot(h, wa[...])
        ab_ref[N:, :] = _dot(h, wb[...])

    out_shape = (jax.ShapeDtypeStruct((N, H), jnp.float32),
                 jax.ShapeDtypeStruct((2 * N, H), jnp.float32))
    return pl.pallas_call(body, out_shape=out_shape)(
        x, *ws, *bs, g, beta, w1a, w1b)


def _edge_enc_call(edge_attr, ws, bs, g, beta, te):
    E = edge_attr.shape[0]
    D = edge_attr.shape[1]
    H = ws[-1].shape[1]
    grid = (E // te,)

    def body(ea_ref, w0, w1, w2, w3, b0, b1, b2, b3, g_ref, bt_ref, out_ref):
        t = jnp.maximum(_dot(ea_ref[...], w0[...]) + b0[...], 0.0)
        t = jnp.maximum(_dot(t, w1[...]) + b1[...], 0.0)
        t = jnp.maximum(_dot(t, w2[...]) + b2[...], 0.0)
        t = _dot(t, w3[...]) + b3[...]
        out_ref[...] = _ln(t, g_ref[...], bt_ref[...])

    full = lambda a: pl.BlockSpec(a.shape, lambda i: (0,) * a.ndim)
    in_specs = [pl.BlockSpec((te, D), lambda i: (i, 0))]
    in_specs += [full(w) for w in ws] + [full(b) for b in bs] + [full(g), full(beta)]
    return pl.pallas_call(
        body,
        grid=grid,
        in_specs=in_specs,
        out_specs=pl.BlockSpec((te, H), lambda i: (i, 0)),
        out_shape=jax.ShapeDtypeStruct((E, H), jnp.float32),
    )(edge_attr, *ws, *bs, g, beta)


def _edge_mlp_call(G, e, w1c, ws, bs, g, beta, te):
    E, H = e.shape
    grid = (E // te,)

    def body(g_in, e_ref, wc, w1, w2, w3, b0, b1, b2, b3, gg, bt, out_ref):
        ev = e_ref[...]
        t = g_in[...] + _dot(ev, wc[...]) + b0[...]
        t = jnp.maximum(t, 0.0)
        t = jnp.maximum(_dot(t, w1[...]) + b1[...], 0.0)
        t = jnp.maximum(_dot(t, w2[...]) + b2[...], 0.0)
        t = _dot(t, w3[...]) + b3[...]
        out_ref[...] = _ln(t, gg[...], bt[...]) + ev

    full = lambda a: pl.BlockSpec(a.shape, lambda i: (0,) * a.ndim)
    row = pl.BlockSpec((te, H), lambda i: (i, 0))
    in_specs = [row, row, full(w1c)] + [full(w) for w in ws]
    in_specs += [full(b) for b in bs] + [full(g), full(beta)]
    return pl.pallas_call(
        body,
        grid=grid,
        in_specs=in_specs,
        out_specs=row,
        out_shape=jax.ShapeDtypeStruct((E, H), jnp.float32),
    )(G, e, w1c, *ws, *bs, g, beta)


def _node_block_call(h, p_a, p_b, v1a, v1b, ws, bs, g, beta, w1a, w1b):
    N, H = h.shape
    with_ab = w1a is not None

    def body(h_ref, pa_ref, pb_ref, va, vb, w1, w2, w3, b0, b1, b2, b3, gg, bt,
             *rest):
        hv = h_ref[...]
        agg = pa_ref[0] + pb_ref[0]
        for c in range(1, p_a.shape[0]):
            agg = agg + pa_ref[c] + pb_ref[c]
        t = _dot(hv, va[...]) + _dot(agg, vb[...]) + b0[...]
        t = jnp.maximum(t, 0.0)
        t = jnp.maximum(_dot(t, w1[...]) + b1[...], 0.0)
        t = jnp.maximum(_dot(t, w2[...]) + b2[...], 0.0)
        t = _dot(t, w3[...]) + b3[...]
        h_new = _ln(t, gg[...], bt[...]) + hv
        if with_ab:
            wa, wb, h_out, ab_out = rest
            h_out[...] = h_new
            ab_out[0:N, :] = _dot(h_new, wa[...])
            ab_out[N:, :] = _dot(h_new, wb[...])
        else:
            (h_out,) = rest
            h_out[...] = h_new

    out_shape = [jax.ShapeDtypeStruct((N, H), jnp.float32)]
    if with_ab:
        out_shape.append(jax.ShapeDtypeStruct((2 * N, H), jnp.float32))
    args = [h, p_a, p_b, v1a, v1b, *ws, *bs, g, beta]
    if with_ab:
        args += [w1a, w1b]
    res = pl.pallas_call(body, out_shape=tuple(out_shape))(*args)
    return res


def _pool_dec_call(h_pad, batch_row, ws, bs, w_last_row, b_last, n_graphs):
    NP = batch_row.shape[1]

    def body(h_ref, batch_ref, w0, w1, w2, b0, b1, b2, wl, bl, out_ref):
        ids = lax.broadcasted_iota(jnp.int32, (n_graphs, NP), 0)
        bm = jnp.broadcast_to(batch_ref[...], (n_graphs, NP))
        onehot = (bm == ids).astype(jnp.float32)
        counts = jnp.sum(onehot, axis=1, keepdims=True)
        sums = jnp.dot(onehot, h_ref[...],
                       preferred_element_type=jnp.float32)
        pooled = sums / jnp.maximum(counts, 1.0)
        t = jnp.maximum(_dot(pooled, w0[...]) + b0[...], 0.0)
        t = jnp.maximum(_dot(t, w1[...]) + b1[...], 0.0)
        t = jnp.maximum(_dot(t, w2[...]) + b2[...], 0.0)
        o = jnp.sum(t * wl[...], axis=-1, keepdims=True) + bl[...]
        out_ref[...] = jax.nn.sigmoid(o)

    return pl.pallas_call(
        body, out_shape=jax.ShapeDtypeStruct((n_graphs, 1), jnp.float32)
    )(h_pad, batch_row, *ws, *bs, w_last_row, b_last)


def _unpack_mlp(mlp):
    layers, ln = mlp
    ws = [W for W, _ in layers]
    bs = [b.reshape(1, -1) for _, b in layers]
    if ln is not None:
        g, beta = ln
        return ws, bs, g.reshape(1, -1), beta.reshape(1, -1)
    return ws, bs, None, None


def kernel(x, edge_index, edge_attr, batch, params):
    N, _ = x.shape
    E = edge_attr.shape[0]
    H = params["node_enc"][0][-1][0].shape[1]
    n_graphs = 16
    mp = len(params["blocks"])

    s = edge_index[0]
    r = edge_index[1]

    ne_ws, ne_bs, ne_g, ne_b = _unpack_mlp(params["node_enc"])
    ee_ws, ee_bs, ee_g, ee_b = _unpack_mlp(params["edge_enc"])
    blocks = []
    for blk in params["blocks"]:
        ews, ebs, eg, eb = _unpack_mlp(blk["edge"])
        w1 = ews[0]
        nws, nbs, ng, nb = _unpack_mlp(blk["node"])
        blocks.append(
            dict(
                w1a=w1[:H], w1b=w1[H:2 * H], w1c=w1[2 * H:],
                e_ws=ews[1:], e_bs=ebs, e_g=eg, e_b=eb,
                v1a=nws[0][:H], v1b=nws[0][H:],
                n_ws=nws[1:], n_bs=nbs, n_g=ng, n_b=nb,
            )
        )
    d_ws, d_bs, _, _ = _unpack_mlp(params["dec"])
    d_last_row = d_ws[-1].reshape(1, -1)
    d_blast = d_bs[-1].reshape(1, 1)

    ch_a, ch_b = 160, 80
    e1 = (E // (2 * NW * ch_a)) * (NW * ch_a)
    e2 = E - e1
    te1 = e1 // 32
    te2 = e2 // 32
    gather_a = _make_gather(e1, H, ch_a, 0)
    gather_b = _make_gather(e2, H, ch_b, 0)
    scat_a = _make_scatter(e1, N, H, ch_b)
    scat_b = _make_scatter(e2, N, H, ch_b)
    s_a, s_b = s[:e1], s[e1:]
    r_a, r_b = r[:e1], r[e1:]

    idx2_a = jnp.stack([s_a.reshape(-1, ch_a), r_a.reshape(-1, ch_a) + N],
                       axis=1).reshape(-1)
    idx2_b = jnp.stack([s_b.reshape(-1, ch_b), r_b.reshape(-1, ch_b) + N],
                       axis=1).reshape(-1)

    h, AB = _node_enc_call(x, ne_ws, ne_bs, ne_g, ne_b,
                           blocks[0]["w1a"], blocks[0]["w1b"])
    e_a = _edge_enc_call(edge_attr[:e1], ee_ws, ee_bs, ee_g, ee_b, te1)
    e_b = _edge_enc_call(edge_attr[e1:], ee_ws, ee_bs, ee_g, ee_b, te2)

    for k in range(mp):
        blk = blocks[k]
        G_a = gather_a(AB, idx2_a)
        G_b = gather_b(AB, idx2_b)
        e_a = _edge_mlp_call(G_a, e_a, blk["w1c"], blk["e_ws"], blk["e_bs"],
                             blk["e_g"], blk["e_b"], te1)
        p_a = scat_a(e_a, r_a)
        e_b = _edge_mlp_call(G_b, e_b, blk["w1c"], blk["e_ws"], blk["e_bs"],
                             blk["e_g"], blk["e_b"], te2)
        p_b = scat_b(e_b, r_b)
        last = k == mp - 1
        if last:
            (h,) = _node_block_call(h, p_a, p_b, blk["v1a"], blk["v1b"],
                                    blk["n_ws"], blk["n_bs"], blk["n_g"],
                                    blk["n_b"], None, None)
        else:
            nxt = blocks[k + 1]
            h, AB = _node_block_call(h, p_a, p_b, blk["v1a"], blk["v1b"],
                                     blk["n_ws"], blk["n_bs"], blk["n_g"],
                                     blk["n_b"], nxt["w1a"], nxt["w1b"])

    n_pad = ((N + 127) // 128) * 128
    h_pad = jnp.pad(h, ((0, n_pad - N), (0, 0)))
    batch_pad = jnp.pad(batch, (0, n_pad - N), constant_values=n_graphs)
    out = _pool_dec_call(h_pad, batch_pad.reshape(1, n_pad),
                         d_ws[:-1], d_bs[:-1], d_last_row, d_blast, n_graphs)
    return out
